# baseline (speedup 1.0000x reference)
import jax
import jax.numpy as jnp
import numpy as np
from jax.experimental import pallas as pl
import jax.experimental.pallas.tpu as pltpu

B, S, D = 2, 2048, 1024
H, DH = 16, 64
E, TOPK, DE = 16, 2, 128
T = B * S

BT = 512
BQ = 256
BK = 256
NEG = -1e30
BF = jnp.bfloat16


def _qkv_kernel(x_ref, r_ref, g_ref, wq_ref, wk_ref, wv_ref, cos_ref, sin_ref,
                q_ref, k_ref, v_ref):
    x = x_ref[...]
    xn = x * r_ref[...] * g_ref[...]
    q = jnp.dot(xn, wq_ref[...], preferred_element_type=jnp.float32)
    k = jnp.dot(xn, wk_ref[...], preferred_element_type=jnp.float32)
    v = jnp.dot(xn, wv_ref[...], preferred_element_type=jnp.float32)
    col = jax.lax.broadcasted_iota(jnp.int32, (1, H * DH), 1)
    cosv = cos_ref[...]
    sinv = sin_ref[...]
    first = (col % 64) < 32
    sgn = jnp.where(first, -sinv, sinv)

    def rope(t):
        partner = jnp.where(first, jnp.roll(t, -32, axis=1), jnp.roll(t, 32, axis=1))
        return t * cosv + partner * sgn

    q_ref[...] = rope(q) * (1.0 / np.sqrt(DH))
    k_ref[...] = rope(k)
    v_ref[...] = v


HG = 4


def _attn_kernel(q_ref, k_ref, v_ref, o_ref, s_ref):
    qi = pl.program_id(2)
    q = q_ref[...].reshape(BQ, HG * DH)
    qs = [q[:, h * DH:(h + 1) * DH] for h in range(HG)]

    nch = ((qi + 1) * BQ + BK - 1) // BK
    rowp = qi * BQ + jax.lax.broadcasted_iota(jnp.int32, (BQ, 1), 0)

    def score_chunk(j, ms):
        kc = k_ref[0, pl.ds(j * BK, BK), :]
        colp = j * BK + jax.lax.broadcasted_iota(jnp.int32, (1, BK), 1)
        maskv = rowp >= colp
        nms = []
        for h in range(HG):
            s = jax.lax.dot_general(qs[h], kc[:, h * DH:(h + 1) * DH],
                                    (((1,), (1,)), ((), ())))
            s = jnp.where(maskv, s, NEG)
            s_ref[h, :, pl.ds(j * BK, BK)] = s
            nms.append(jnp.maximum(ms[h], jnp.max(s, axis=1, keepdims=True)))
        return nms

    ms = jax.lax.fori_loop(0, nch, score_chunk,
                           [jnp.full((BQ, 1), NEG, jnp.float32)] * HG)

    def expsum_chunk(j, lsc):
        nls = []
        for h in range(HG):
            p = jnp.exp(s_ref[h, :, pl.ds(j * BK, BK)] - ms[h])
            s_ref[h, :, pl.ds(j * BK, BK)] = p
            nls.append(lsc[h] + jnp.sum(p, axis=1, keepdims=True))
        return nls

    ls = jax.lax.fori_loop(0, nch, expsum_chunk,
                           [jnp.zeros((BQ, 1), jnp.float32)] * HG)

    def pv_chunk(j, accs):
        vc = v_ref[0, pl.ds(j * BK, BK), :]
        naccs = []
        for h in range(HG):
            attn = s_ref[h, :, pl.ds(j * BK, BK)] / ls[h]
            naccs.append(accs[h] + jax.lax.dot_general(
                attn, vc[:, h * DH:(h + 1) * DH], (((1,), (0,)), ((), ()))))
        return naccs

    accs = jax.lax.fori_loop(0, nch, pv_chunk,
                             [jnp.zeros((BQ, DH), jnp.float32)] * HG)
    o_ref[...] = jnp.concatenate(accs, axis=1).reshape(1, BQ, HG * DH)


def _proj_kernel(x_ref, ctx_ref, wo_ref, xres_ref):
    xres_ref[...] = x_ref[...] + jnp.dot(ctx_ref[...], wo_ref[...],
                                         preferred_element_type=jnp.float32)


def _router_kernel(xres_ref, r_ref, g_ref, wr_ref, xn2_ref, gates_ref):
    xn2 = xres_ref[...] * r_ref[...] * g_ref[...]
    logits = jnp.dot(xn2, wr_ref[...], preferred_element_type=jnp.float32)
    m = jnp.max(logits, axis=1, keepdims=True)
    pe = jnp.exp(logits - m)
    probs = pe / jnp.sum(pe, axis=1, keepdims=True)
    lane = jax.lax.broadcasted_iota(jnp.int32, (BT, E), 1)
    m1 = jnp.max(probs, axis=1, keepdims=True)
    i1 = jnp.min(jnp.where(probs == m1, lane, E), axis=1, keepdims=True)
    pm = jnp.where(lane == i1, -1.0, probs)
    m2 = jnp.max(pm, axis=1, keepdims=True)
    i2 = jnp.min(jnp.where(pm == m2, lane, E), axis=1, keepdims=True)
    denom = (m1 + m2) + 1e-9
    gates = jnp.where(lane == i1, m1 / denom,
                      jnp.where(lane == i2, m2 / denom, 0.0))
    xn2_ref[...] = xn2.astype(BF)
    gates_ref[...] = gates


def _moe_kernel(xn2_ref, gates_ref, w1_ref, w2_ref, xres_ref, out_ref):
    p = pl.program_id(1)
    xb = xn2_ref[...]
    h = jnp.dot(xb, w1_ref[0], preferred_element_type=jnp.float32)
    h = jnp.maximum(h, 0.0)
    lane = jax.lax.broadcasted_iota(jnp.int32, (BT, E), 1)
    g = gates_ref[...]
    g1 = jnp.sum(jnp.where(lane == 2 * p, g, 0.0), axis=1, keepdims=True)
    g2 = jnp.sum(jnp.where(lane == 2 * p + 1, g, 0.0), axis=1, keepdims=True)
    lane2 = jax.lax.broadcasted_iota(jnp.int32, (BT, 2 * DE), 1)
    gh = (jnp.where(lane2 < DE, g1, g2) * h).astype(BF)
    o = jnp.dot(gh, w2_ref[0], preferred_element_type=jnp.float32)

    @pl.when(p == 0)
    def _():
        out_ref[...] = xres_ref[...] + o

    @pl.when(p > 0)
    def _():
        out_ref[...] += o


def kernel(token_stream, g_attn_pre, g_ffn_pre, Wq, Wk, Wv, Wo, Wr, W1, W2):
    x = token_stream.reshape(T, D)
    ga = g_attn_pre.reshape(1, D)
    gf = g_ffn_pre.reshape(1, D)
    wq, wk, wv, wo = Wq, Wk, Wv, Wo
    w1r = W1.reshape(E // 2, 2, D, DE).transpose(0, 2, 1, 3) \
            .reshape(E // 2, D, 2 * DE).astype(BF)
    w2r = W2.reshape(E // 2, 2 * DE, D).astype(BF)
    half = DH // 2
    inv_freq = 1.0 / (10000.0 ** (jnp.arange(half, dtype=jnp.float32) / half))
    angv = jnp.arange(S, dtype=jnp.float32)[:, None] * inv_freq[None, :]
    cos_t = jnp.tile(jnp.cos(angv), (1, H * DH // half))
    sin_t = jnp.tile(jnp.sin(angv), (1, H * DH // half))

    r1 = jax.lax.rsqrt(jnp.mean(x * x, axis=-1, keepdims=True) + 1e-6)

    q, k, v = pl.pallas_call(
        _qkv_kernel,
        grid=(T // BT,),
        in_specs=[
            pl.BlockSpec((BT, D), lambda i: (i, 0)),
            pl.BlockSpec((BT, 1), lambda i: (i, 0)),
            pl.BlockSpec((1, D), lambda i: (0, 0)),
            pl.BlockSpec((D, D), lambda i: (0, 0)),
            pl.BlockSpec((D, D), lambda i: (0, 0)),
            pl.BlockSpec((D, D), lambda i: (0, 0)),
            pl.BlockSpec((BT, H * DH), lambda i: (i % (S // BT), 0)),
            pl.BlockSpec((BT, H * DH), lambda i: (i % (S // BT), 0)),
        ],
        out_specs=[pl.BlockSpec((BT, D), lambda i: (i, 0))] * 3,
        out_shape=[jax.ShapeDtypeStruct((T, D), jnp.float32)] * 3,
    )(x, r1, ga, wq, wk, wv, cos_t, sin_t)

    q3 = q.reshape(B, S, H * DH)
    k3 = k.reshape(B, S, H * DH)
    v3 = v.reshape(B, S, H * DH)
    ctx = pl.pallas_call(
        _attn_kernel,
        grid=(B, H // HG, S // BQ),
        in_specs=[
            pl.BlockSpec((1, BQ, HG * DH), lambda b, g, i: (b, i, g)),
            pl.BlockSpec((1, S, HG * DH), lambda b, g, i: (b, 0, g)),
            pl.BlockSpec((1, S, HG * DH), lambda b, g, i: (b, 0, g)),
        ],
        out_specs=pl.BlockSpec((1, BQ, HG * DH), lambda b, g, i: (b, i, g)),
        out_shape=jax.ShapeDtypeStruct((B, S, H * DH), jnp.float32),
        scratch_shapes=[pltpu.VMEM((HG, BQ, S), jnp.float32)],
    )(q3, k3, v3)

    xres = pl.pallas_call(
        _proj_kernel,
        grid=(T // BT,),
        in_specs=[
            pl.BlockSpec((BT, D), lambda i: (i, 0)),
            pl.BlockSpec((BT, D), lambda i: (i, 0)),
            pl.BlockSpec((D, D), lambda i: (0, 0)),
        ],
        out_specs=pl.BlockSpec((BT, D), lambda i: (i, 0)),
        out_shape=jax.ShapeDtypeStruct((T, D), jnp.float32),
    )(x, ctx.reshape(T, H * DH), wo)

    r2 = jax.lax.rsqrt(jnp.mean(xres * xres, axis=-1, keepdims=True) + 1e-6)

    xn2, gates = pl.pallas_call(
        _router_kernel,
        grid=(T // BT,),
        in_specs=[
            pl.BlockSpec((BT, D), lambda i: (i, 0)),
            pl.BlockSpec((BT, 1), lambda i: (i, 0)),
            pl.BlockSpec((1, D), lambda i: (0, 0)),
            pl.BlockSpec((D, E), lambda i: (0, 0)),
        ],
        out_specs=[
            pl.BlockSpec((BT, D), lambda i: (i, 0)),
            pl.BlockSpec((BT, E), lambda i: (i, 0)),
        ],
        out_shape=[
            jax.ShapeDtypeStruct((T, D), BF),
            jax.ShapeDtypeStruct((T, E), jnp.float32),
        ],
    )(xres, r2, gf, Wr)

    out = pl.pallas_call(
        _moe_kernel,
        grid=(T // BT, E // 2),
        in_specs=[
            pl.BlockSpec((BT, D), lambda t, p: (t, 0)),
            pl.BlockSpec((BT, E), lambda t, p: (t, 0)),
            pl.BlockSpec((1, D, 2 * DE), lambda t, p: (p, 0, 0)),
            pl.BlockSpec((1, 2 * DE, D), lambda t, p: (p, 0, 0)),
            pl.BlockSpec((BT, D), lambda t, p: (t, 0)),
        ],
        out_specs=pl.BlockSpec((BT, D), lambda t, p: (t, 0)),
        out_shape=jax.ShapeDtypeStruct((T, D), jnp.float32),
    )(xn2, gates, w1r, w2r, xres)

    return out.reshape(B, S, D)
